# baseline (speedup 1.0000x reference)
import functools

import jax
import jax.numpy as jnp
from jax import lax
from jax.experimental import pallas as pl
from jax.experimental.pallas import tpu as pltpu
from jax.experimental.pallas import tpu_sc as plsc

MAX_SPEAKERS = 1000000
EMBED_DIM = 64
HIDDEN_SIZE = 1024
BATCH = 16384
_L = 16
_T = (MAX_SPEAKERS + 127) // 128
_NBUF = 4
_C = 640
_SEL = 720
_ICH = 2048
_PADB = BATCH
_X2R = BATCH + _ICH


def _make_sc_gather(B):
    info = plsc.get_sparse_core_info()
    NC, NS = info.num_cores, info.num_subcores
    NW = NC * NS
    mesh = plsc.VectorSubcoreMesh(core_axis_name="c", subcore_axis_name="s")
    n_ich = B // _ICH

    @functools.partial(
        pl.kernel,
        mesh=mesh,
        out_type=jax.ShapeDtypeStruct((_X2R, 2 * EMBED_DIM), jnp.float32),
        scratch_types=[
            pltpu.VMEM((2, _ICH), jnp.int32),
            pltpu.VMEM((_SEL,), jnp.int32),
            pltpu.VMEM((_SEL,), jnp.int32),
            pltpu.VMEM((_C // 128, 128), jnp.int32),
            pltpu.VMEM((_C, 2 * EMBED_DIM), jnp.float32),
            pltpu.VMEM((_NBUF, EMBED_DIM, 128), jnp.float32),
            pltpu.VMEM((32,), jnp.int32),
            pltpu.VMEM((32,), jnp.int32),
            pltpu.VMEM((_L,), jnp.int32),
            pltpu.VMEM((_L, 2 * EMBED_DIM), jnp.float32),
            pltpu.SemaphoreType.DMA,
            pltpu.SemaphoreType.DMA,
            pltpu.SemaphoreType.DMA,
        ],
        compiler_params=pltpu.CompilerParams(needs_layout_passes=False),
    )
    def gather_kernel(
        tableT_hbm,
        idx_hbm,
        x2_hbm,
        ibuf,
        sel_id,
        sel_pos,
        pos2d,
        rows_v,
        slabs,
        tmp_id,
        tmp_j,
        ov_pos,
        ov_row,
        sem_i,
        sem_s,
        sem_o,
    ):
        wid = lax.axis_index("s") * NC + lax.axis_index("c")
        lo = (wid * _T) // NW
        hi = ((wid + 1) * _T) // NW
        ntc = hi - lo
        iota = lax.iota(jnp.int32, _L)
        pad16 = _PADB + wid * 4 + (iota & 3)
        cvec = [iota + k * _L for k in range(4)]

        def scalar_at(ref, i):
            return ref[pl.ds(i, _L)][0]

        def prefill(k, _):
            sel_pos[pl.ds(k * _L, _L)] = pad16
            return ()

        lax.fori_loop(0, _SEL // _L, prefill, (), unroll=False)

        pltpu.async_copy(idx_hbm.at[pl.ds(0, _ICH)], ibuf.at[0], sem_i).wait()

        def scan_chunk(c, off):
            buf = lax.rem(c, 2)

            @pl.when(c + 1 < n_ich)
            def _():
                pltpu.async_copy(
                    idx_hbm.at[pl.ds((c + 1) * _ICH, _ICH)],
                    ibuf.at[lax.rem(c + 1, 2)],
                    sem_i,
                )

            def scan16(g, off):
                v = ibuf[buf, pl.ds(g * _L, _L)]
                tc16 = lax.shift_right_logical(v, 7)
                inr = (tc16 >= lo) & (tc16 < hi)
                p16 = c * _ICH + g * _L + iota
                plsc.store_compressed(sel_id.at[pl.ds(off, _L)], v, mask=inr)
                plsc.store_compressed(sel_pos.at[pl.ds(off, _L)], p16, mask=inr)
                cnt = plsc.all_reduce_population_count(inr)[0]
                return off + cnt

            off = lax.fori_loop(0, _ICH // _L, scan16, off, unroll=False)

            @pl.when(c + 1 < n_ich)
            def _():
                pltpu.make_async_copy(
                    idx_hbm.at[pl.ds(0, _ICH)], ibuf.at[0], sem_i
                ).wait()

            return off

        n_sel = lax.fori_loop(0, n_ich, scan_chunk, 0, unroll=False)
        nch = lax.div(n_sel + _L - 1, _L)

        def poscopy(k, _):
            pos2d[lax.div(k, 8), pl.ds(lax.rem(k, 8) * _L, _L)] = sel_pos[
                pl.ds(k * _L, _L)
            ]
            return ()

        lax.fori_loop(0, _C // _L, poscopy, (), unroll=False)

        def fetch(t, slot):
            pltpu.async_copy(
                tableT_hbm.at[:, pl.ds((lo + t) * 128, 128)],
                slabs.at[slot],
                sem_s,
            )

        for p in range(_NBUF - 1):
            fetch(p, p)

        def extract_to(dst_ref, jrow, lane, slot):
            lvec = jnp.full((_L,), lane, jnp.int32)
            for k4 in range(4):
                vals = plsc.load_gather(slabs.at[slot], [cvec[k4], lvec])
                dst_ref[jrow, pl.ds(k4 * _L, _L)] = vals

        def tc_body(t, _):
            slot = lax.rem(t, _NBUF)
            pltpu.make_async_copy(
                tableT_hbm.at[:, pl.ds(0, 128)], slabs.at[slot], sem_s
            ).wait()

            @pl.when(t + _NBUF - 1 < ntc)
            def _():
                fetch(t + _NBUF - 1, lax.rem(t + _NBUF - 1, _NBUF))

            cur = lo + t

            def match16(c, _):
                vid = sel_id[pl.ds(c * _L, _L)]
                cp16 = c * _L + iota
                m = (lax.shift_right_logical(vid, 7) == cur) & (cp16 < n_sel)
                plsc.store_compressed(tmp_id.at[pl.ds(0, _L)], vid, mask=m)
                plsc.store_compressed(tmp_j.at[pl.ds(0, _L)], cp16, mask=m)
                cnt = plsc.all_reduce_population_count(m)[0]

                def emit(k, _):
                    idk = scalar_at(tmp_id, k)
                    jk = scalar_at(tmp_j, k)
                    lane = idk & 127

                    @pl.when(jk < _C)
                    def _():
                        extract_to(rows_v, jk, lane, slot)

                    @pl.when(jk >= _C)
                    def _():
                        extract_to(ov_row, 0, lane, slot)
                        pk = scalar_at(sel_pos, jk)
                        ov_pos[pl.ds(0, _L)] = jnp.where(iota == 0, pk, pad16)
                        pltpu.async_copy(
                            ov_row, x2_hbm.at[ov_pos], sem_o
                        ).wait()

                    return ()

                lax.fori_loop(0, cnt, emit, (), unroll=False)
                return ()

            lax.fori_loop(0, nch, match16, (), unroll=False)
            return ()

        lax.fori_loop(0, ntc, tc_body, (), unroll=False)

        for g in range(_C // 128):
            pltpu.async_copy(
                rows_v.at[pl.ds(g * 128, 128)], x2_hbm.at[pos2d.at[g]], sem_o
            )
        for g in range(_C // 128):
            pltpu.make_async_copy(
                rows_v.at[pl.ds(0, 128)], x2_hbm.at[pos2d.at[0]], sem_o
            ).wait()

    return gather_kernel


def _proj_body(x2_ref, wt_ref, b_ref, o_ref):
    o_ref[...] = (
        lax.dot_general(
            x2_ref[:, :EMBED_DIM],
            wt_ref[...],
            (((1,), (0,)), ((), ())),
            preferred_element_type=jnp.float32,
        )
        + b_ref[...]
    )


_BB = 2048


def _make_tc_proj(B, H):
    return pl.pallas_call(
        _proj_body,
        grid=(B // _BB,),
        in_specs=[
            pl.BlockSpec((_BB, 2 * EMBED_DIM), lambda i: (i, 0)),
            pl.BlockSpec((EMBED_DIM, H), lambda i: (0, 0)),
            pl.BlockSpec((1, H), lambda i: (0, 0)),
        ],
        out_specs=pl.BlockSpec((_BB, H), lambda i: (i, 0)),
        out_shape=jax.ShapeDtypeStruct((B, H), jnp.float32),
    )


@jax.jit
def kernel(speaker_ids, table, W, b):
    ids = speaker_ids.astype(jnp.int32)
    gather = _make_sc_gather(BATCH)
    x2 = gather(table.T, ids)
    proj = _make_tc_proj(BATCH, HIDDEN_SIZE)
    return proj(x2, W.T, b.reshape(1, HIDDEN_SIZE))


# speedup vs baseline: 1.3407x; 1.3407x over previous
import functools

import jax
import jax.numpy as jnp
from jax import lax
from jax.experimental import pallas as pl
from jax.experimental.pallas import tpu as pltpu
from jax.experimental.pallas import tpu_sc as plsc

MAX_SPEAKERS = 1000000
EMBED_DIM = 64
HIDDEN_SIZE = 1024
BATCH = 16384
_L = 16
_NBUF = 6


def _make_sc_gather(B):
    info = plsc.get_sparse_core_info()
    NC, NS = info.num_cores, info.num_subcores
    NW = NC * NS
    b_per_w = B // NW
    mesh = plsc.VectorSubcoreMesh(core_axis_name="c", subcore_axis_name="s")

    @functools.partial(
        pl.kernel,
        mesh=mesh,
        out_type=jax.ShapeDtypeStruct((B, 2 * EMBED_DIM), jnp.float32),
        scratch_types=[
            pltpu.VMEM((b_per_w + _L,), jnp.int32),
            pltpu.VMEM((_NBUF, EMBED_DIM, 2 * EMBED_DIM), jnp.float32),
            pltpu.VMEM((b_per_w, 2 * EMBED_DIM), jnp.float32),
            pltpu.SemaphoreType.DMA,
        ],
        compiler_params=pltpu.CompilerParams(needs_layout_passes=False),
    )
    def gather_kernel(tableT_hbm, idx_hbm, x2_hbm, idx_v, slab_v, rows_v, sem):
        wid = lax.axis_index("s") * NC + lax.axis_index("c")
        base = wid * b_per_w
        pltpu.sync_copy(idx_hbm.at[pl.ds(base, b_per_w)], idx_v.at[pl.ds(0, b_per_w)])

        cvec = [lax.iota(jnp.int32, _L) + k * _L for k in range(4)]

        def fetch(scalar_id, buf):
            tc = lax.shift_right_logical(scalar_id, 7) * 128
            pltpu.async_copy(
                tableT_hbm.at[:, pl.ds(tc, 128)], slab_v.at[buf], sem
            )

        def extract(j, scalar_id, buf):
            lvec = jnp.full((_L,), scalar_id & 127, jnp.int32)
            for k in range(4):
                vals = plsc.load_gather(slab_v.at[buf], [cvec[k], lvec])
                rows_v[j, pl.ds(k * _L, _L)] = vals

        v0 = idx_v[pl.ds(0, _L)]
        for b in range(_NBUF):
            fetch(v0[b], b)

        n_blocks = b_per_w // _L

        def block_body(r, _):
            j0 = r * _L
            v_cur = idx_v[pl.ds(j0, _L)]
            v_nxt = idx_v[pl.ds(j0 + _L, _L)]
            for t in range(_L):
                j = j0 + t
                buf = (
                    lax.rem(j, _NBUF)
                    if _L % _NBUF
                    else t % _NBUF
                )
                pltpu.make_async_copy(
                    tableT_hbm.at[:, pl.ds(0, 128)], slab_v.at[buf], sem
                ).wait()
                extract(j, v_cur[t], buf)
                nid = (
                    v_cur[t + _NBUF] if t + _NBUF < _L else v_nxt[t + _NBUF - _L]
                )

                @pl.when(j + _NBUF < b_per_w)
                def _():
                    fetch(nid, buf)

            return ()

        lax.fori_loop(0, n_blocks, block_body, (), unroll=False)
        pltpu.sync_copy(rows_v, x2_hbm.at[pl.ds(base, b_per_w)])

    return gather_kernel


def _proj_body(x2_ref, wt_ref, b_ref, o_ref):
    o_ref[...] = (
        lax.dot_general(
            x2_ref[:, :EMBED_DIM],
            wt_ref[...],
            (((1,), (0,)), ((), ())),
            preferred_element_type=jnp.float32,
        )
        + b_ref[...]
    )


def _proj_body_acc(x2_ref, wt_ref, b_ref, prev_ref, o_ref):
    del prev_ref
    _proj_body(x2_ref, wt_ref, b_ref, o_ref)


_NCHUNK = 1
_BB = 2048


def _make_tc_proj(B, H, chunk):
    BC = B // _NCHUNK
    r0 = (chunk * BC) // _BB
    in_specs = [
        pl.BlockSpec((_BB, 2 * EMBED_DIM), lambda i: (i, 0)),
        pl.BlockSpec((EMBED_DIM, H), lambda i: (0, 0)),
        pl.BlockSpec((1, H), lambda i: (0, 0)),
    ]
    kwargs = {}
    body = _proj_body
    if chunk > 0:
        in_specs.append(pl.BlockSpec(memory_space=pl.ANY))
        kwargs["input_output_aliases"] = {3: 0}
        body = _proj_body_acc
    return pl.pallas_call(
        body,
        grid=(BC // _BB,),
        in_specs=in_specs,
        out_specs=pl.BlockSpec((_BB, H), lambda i: (r0 + i, 0)),
        out_shape=jax.ShapeDtypeStruct((B, H), jnp.float32),
        **kwargs,
    )


@jax.jit
def kernel(speaker_ids, table, W, b):
    ids = speaker_ids.astype(jnp.int32)
    tableT = table.T
    Wt = W.T
    b2 = b.reshape(1, HIDDEN_SIZE)
    BC = BATCH // _NCHUNK
    gather = _make_sc_gather(BC)
    x2s = [
        gather(tableT, lax.slice(ids, (c * BC,), ((c + 1) * BC,)))
        for c in range(_NCHUNK)
    ]
    out = _make_tc_proj(BATCH, HIDDEN_SIZE, 0)(x2s[0], Wt, b2)
    for c in range(1, _NCHUNK):
        out = _make_tc_proj(BATCH, HIDDEN_SIZE, c)(x2s[c], Wt, b2, out)
    return out


# speedup vs baseline: 1.3442x; 1.0026x over previous
import functools

import jax
import jax.numpy as jnp
from jax import lax
from jax.experimental import pallas as pl
from jax.experimental.pallas import tpu as pltpu
from jax.experimental.pallas import tpu_sc as plsc

MAX_SPEAKERS = 1000000
EMBED_DIM = 64
HIDDEN_SIZE = 1024
BATCH = 16384
_L = 16
_NBUF = 7


def _make_sc_gather(B):
    info = plsc.get_sparse_core_info()
    NC, NS = info.num_cores, info.num_subcores
    NW = NC * NS
    b_per_w = B // NW
    mesh = plsc.VectorSubcoreMesh(core_axis_name="c", subcore_axis_name="s")

    @functools.partial(
        pl.kernel,
        mesh=mesh,
        out_type=jax.ShapeDtypeStruct((B, 2 * EMBED_DIM), jnp.float32),
        scratch_types=[
            pltpu.VMEM((b_per_w + _L,), jnp.int32),
            pltpu.VMEM((_NBUF, EMBED_DIM, 2 * EMBED_DIM), jnp.float32),
            pltpu.VMEM((b_per_w, 2 * EMBED_DIM), jnp.float32),
            pltpu.SemaphoreType.DMA,
        ],
        compiler_params=pltpu.CompilerParams(needs_layout_passes=False),
    )
    def gather_kernel(tableT_hbm, idx_hbm, x2_hbm, idx_v, slab_v, rows_v, sem):
        wid = lax.axis_index("s") * NC + lax.axis_index("c")
        base = wid * b_per_w
        pltpu.sync_copy(idx_hbm.at[pl.ds(base, b_per_w)], idx_v.at[pl.ds(0, b_per_w)])

        cvec = [lax.iota(jnp.int32, _L) + k * _L for k in range(4)]

        def fetch(scalar_id, buf):
            tc = lax.shift_right_logical(scalar_id, 7) * 128
            pltpu.async_copy(
                tableT_hbm.at[:, pl.ds(tc, 128)], slab_v.at[buf], sem
            )

        def extract(j, scalar_id, buf):
            lvec = jnp.full((_L,), scalar_id & 127, jnp.int32)
            for k in range(4):
                vals = plsc.load_gather(slab_v.at[buf], [cvec[k], lvec])
                rows_v[j, pl.ds(k * _L, _L)] = vals

        v0 = idx_v[pl.ds(0, _L)]
        for b in range(_NBUF):
            fetch(v0[b], b)

        n_blocks = b_per_w // _L

        def block_body(r, _):
            j0 = r * _L
            v_cur = idx_v[pl.ds(j0, _L)]
            v_nxt = idx_v[pl.ds(j0 + _L, _L)]
            for t in range(_L):
                j = j0 + t
                buf = (
                    lax.rem(j, _NBUF)
                    if _L % _NBUF
                    else t % _NBUF
                )
                pltpu.make_async_copy(
                    tableT_hbm.at[:, pl.ds(0, 128)], slab_v.at[buf], sem
                ).wait()
                extract(j, v_cur[t], buf)
                nid = (
                    v_cur[t + _NBUF] if t + _NBUF < _L else v_nxt[t + _NBUF - _L]
                )

                @pl.when(j + _NBUF < b_per_w)
                def _():
                    fetch(nid, buf)

            return ()

        lax.fori_loop(0, n_blocks, block_body, (), unroll=False)
        pltpu.sync_copy(rows_v, x2_hbm.at[pl.ds(base, b_per_w)])

    return gather_kernel


def _proj_body(x2_ref, wt_ref, b_ref, o_ref):
    o_ref[...] = (
        lax.dot_general(
            x2_ref[:, :EMBED_DIM],
            wt_ref[...],
            (((1,), (0,)), ((), ())),
            preferred_element_type=jnp.float32,
        )
        + b_ref[...]
    )


def _proj_body_acc(x2_ref, wt_ref, b_ref, prev_ref, o_ref):
    del prev_ref
    _proj_body(x2_ref, wt_ref, b_ref, o_ref)


_NCHUNK = 1
_BB = 2048


def _make_tc_proj(B, H, chunk):
    BC = B // _NCHUNK
    r0 = (chunk * BC) // _BB
    in_specs = [
        pl.BlockSpec((_BB, 2 * EMBED_DIM), lambda i: (i, 0)),
        pl.BlockSpec((EMBED_DIM, H), lambda i: (0, 0)),
        pl.BlockSpec((1, H), lambda i: (0, 0)),
    ]
    kwargs = {}
    body = _proj_body
    if chunk > 0:
        in_specs.append(pl.BlockSpec(memory_space=pl.ANY))
        kwargs["input_output_aliases"] = {3: 0}
        body = _proj_body_acc
    return pl.pallas_call(
        body,
        grid=(BC // _BB,),
        in_specs=in_specs,
        out_specs=pl.BlockSpec((_BB, H), lambda i: (r0 + i, 0)),
        out_shape=jax.ShapeDtypeStruct((B, H), jnp.float32),
        **kwargs,
    )


@jax.jit
def kernel(speaker_ids, table, W, b):
    ids = speaker_ids.astype(jnp.int32)
    tableT = table.T
    Wt = W.T
    b2 = b.reshape(1, HIDDEN_SIZE)
    BC = BATCH // _NCHUNK
    gather = _make_sc_gather(BC)
    x2s = [
        gather(tableT, lax.slice(ids, (c * BC,), ((c + 1) * BC,)))
        for c in range(_NCHUNK)
    ]
    out = _make_tc_proj(BATCH, HIDDEN_SIZE, 0)(x2s[0], Wt, b2)
    for c in range(1, _NCHUNK):
        out = _make_tc_proj(BATCH, HIDDEN_SIZE, c)(x2s[c], Wt, b2, out)
    return out
